# Initial kernel scaffold; baseline (speedup 1.0000x reference)
import jax
import jax.numpy as jnp
from jax.experimental import pallas as pl


def kernel(inputs, task_embeddings):
    raise NotImplementedError("write your pallas kernel here")


# speedup vs baseline: 1.3987x; 1.3987x over previous
import jax
import jax.numpy as jnp
from jax.experimental import pallas as pl


def _nn_kernel(x_ref, emb_ref, out_ref):
    x = x_ref[...]
    xn = x * jax.lax.rsqrt(jnp.sum(x * x, axis=-1, keepdims=True))
    e = emb_ref[0]
    d = jax.lax.dot_general(
        xn, e, (((1,), (1,)), ((), ())),
        preferred_element_type=jnp.float32)
    maxd = jnp.max(d, axis=1)
    out_ref[0, 0, :] = -jnp.sqrt(jnp.maximum(2.0 - 2.0 * maxd, 0.0))


def kernel(inputs, task_embeddings):
    B, D = inputs.shape
    T, S, _ = task_embeddings.shape
    out = pl.pallas_call(
        _nn_kernel,
        grid=(T,),
        in_specs=[
            pl.BlockSpec((B, D), lambda t: (0, 0)),
            pl.BlockSpec((1, S, D), lambda t: (t, 0, 0)),
        ],
        out_specs=pl.BlockSpec((1, 1, B), lambda t: (t, 0, 0)),
        out_shape=jax.ShapeDtypeStruct((T, 1, B), jnp.float32),
    )(inputs, task_embeddings)
    return out[:, 0, :].T


# speedup vs baseline: 2.5243x; 1.8048x over previous
import jax
import jax.numpy as jnp
from jax.experimental import pallas as pl
from jax.experimental.pallas import tpu as pltpu


def _nn_kernel(x_ref, emb_ref, out_ref, xn_ref):
    @pl.when(pl.program_id(0) == 0)
    def _():
        x = x_ref[...]
        xn_ref[...] = x * jax.lax.rsqrt(
            jnp.sum(x * x, axis=-1, keepdims=True))

    e = emb_ref[0]
    d = jax.lax.dot_general(
        e, xn_ref[...], (((1,), (1,)), ((), ())),
        preferred_element_type=jnp.float32)
    maxd = jnp.max(d, axis=0)
    out_ref[0, 0, :] = -jnp.sqrt(jnp.maximum(2.0 - 2.0 * maxd, 0.0))


def kernel(inputs, task_embeddings):
    B, D = inputs.shape
    T, S, _ = task_embeddings.shape
    out = pl.pallas_call(
        _nn_kernel,
        grid=(T,),
        in_specs=[
            pl.BlockSpec((B, D), lambda t: (0, 0)),
            pl.BlockSpec((1, S, D), lambda t: (t, 0, 0)),
        ],
        out_specs=pl.BlockSpec((1, 1, B), lambda t: (t, 0, 0)),
        out_shape=jax.ShapeDtypeStruct((T, 1, B), jnp.float32),
        scratch_shapes=[pltpu.VMEM((B, D), jnp.float32)],
    )(inputs, task_embeddings)
    return out[:, 0, :].T


# speedup vs baseline: 3.3141x; 1.3129x over previous
import jax
import jax.numpy as jnp
from jax.experimental import pallas as pl
from jax.experimental.pallas import tpu as pltpu

_TB = 10


def _nn_kernel(x_ref, emb_ref, out_ref, xn_ref):
    @pl.when(pl.program_id(0) == 0)
    def _():
        x = x_ref[...]
        xn_ref[...] = x * jax.lax.rsqrt(
            jnp.sum(x * x, axis=-1, keepdims=True))

    xn = xn_ref[...]
    for t in range(_TB):
        e = emb_ref[t]
        d = jax.lax.dot_general(
            e, xn, (((1,), (1,)), ((), ())),
            preferred_element_type=jnp.float32)
        maxd = jnp.max(d, axis=0)
        out_ref[0, t, :] = -jnp.sqrt(jnp.maximum(2.0 - 2.0 * maxd, 0.0))


def kernel(inputs, task_embeddings):
    B, D = inputs.shape
    T, S, _ = task_embeddings.shape
    nblk = T // _TB
    out = pl.pallas_call(
        _nn_kernel,
        grid=(nblk,),
        in_specs=[
            pl.BlockSpec((B, D), lambda t: (0, 0)),
            pl.BlockSpec((_TB, S, D), lambda t: (t, 0, 0)),
        ],
        out_specs=pl.BlockSpec((1, _TB, B), lambda t: (t, 0, 0)),
        out_shape=jax.ShapeDtypeStruct((nblk, _TB, B), jnp.float32),
        scratch_shapes=[pltpu.VMEM((B, D), jnp.float32)],
    )(inputs, task_embeddings)
    return out.reshape(T, B).T


# speedup vs baseline: 5.0404x; 1.5209x over previous
import jax
import jax.numpy as jnp
from jax.experimental import pallas as pl
from jax.experimental.pallas import tpu as pltpu

_TB = 10


def _nn_kernel(x_ref, emb_ref, out_ref, xn_ref):
    @pl.when(pl.program_id(0) == 0)
    def _():
        x = x_ref[...]
        xn_ref[...] = x * jax.lax.rsqrt(
            jnp.sum(x * x, axis=-1, keepdims=True))

    xn = xn_ref[...]
    for t in range(_TB):
        et = emb_ref[t]
        d = jax.lax.dot_general(
            et, xn, (((0,), (1,)), ((), ())),
            preferred_element_type=jnp.float32)
        maxd = jnp.max(d, axis=0)
        out_ref[0, t, :] = -jnp.sqrt(jnp.maximum(2.0 - 2.0 * maxd, 0.0))


def kernel(inputs, task_embeddings):
    B, D = inputs.shape
    T, S, _ = task_embeddings.shape
    nblk = T // _TB
    emb_t = task_embeddings.transpose(0, 2, 1)
    out = pl.pallas_call(
        _nn_kernel,
        grid=(nblk,),
        in_specs=[
            pl.BlockSpec((B, D), lambda t: (0, 0)),
            pl.BlockSpec((_TB, D, S), lambda t: (t, 0, 0)),
        ],
        out_specs=pl.BlockSpec((1, _TB, B), lambda t: (t, 0, 0)),
        out_shape=jax.ShapeDtypeStruct((nblk, _TB, B), jnp.float32),
        scratch_shapes=[pltpu.VMEM((B, D), jnp.float32)],
    )(inputs, emb_t)
    return out.reshape(T, B).T
